# baseline (speedup 1.0000x reference)
import functools

import jax
import jax.numpy as jnp
from jax.experimental import pallas as pl
from jax.experimental.pallas import tpu as pltpu
from jax.experimental.pallas import tpu_sc as plsc

R = 128
V = 100000
RB = 8
NSTEP = R // RB


def _argmax_zero_body(p_ref, z_ref, idx_ref):
    x = p_ref[...]
    m = jnp.max(x, axis=1, keepdims=True)
    col = jax.lax.broadcasted_iota(jnp.int32, (RB, V), 1)
    cand = jnp.where(x == m, col, jnp.int32(2**31 - 1))
    first = jnp.min(cand, axis=1, keepdims=True)
    j = pl.program_id(0)
    row = jax.lax.broadcasted_iota(jnp.int32, (RB, 1), 0) + j * RB
    idx_ref[...] = first + row * V
    z_ref[...] = jnp.zeros((RB, V), jnp.float32)


_argmax_zero = pl.pallas_call(
    _argmax_zero_body,
    grid=(NSTEP,),
    in_specs=[pl.BlockSpec((RB, V), lambda j: (j, 0))],
    out_specs=[
        pl.BlockSpec((RB, V), lambda j: (j, 0)),
        pl.BlockSpec((RB, 1), lambda j: (j, 0)),
    ],
    out_shape=[
        jax.ShapeDtypeStruct((R, V), jnp.float32),
        jax.ShapeDtypeStruct((R, 1), jnp.int32),
    ],
)


@functools.cache
def _make_sc_scatter_ones():
    @functools.partial(
        pl.kernel,
        out_type=(),
        mesh=plsc.VectorSubcoreMesh(core_axis_name="c", subcore_axis_name="s"),
        scratch_types=[
            pltpu.VMEM((R,), jnp.int32),
            pltpu.VMEM((R,), jnp.float32),
            pltpu.SemaphoreType.DMA,
        ],
    )
    def _sc_scatter_ones(idx_hbm, out_hbm, idx_v, ones_v, sem):
        cid = jax.lax.axis_index("c")
        sid = jax.lax.axis_index("s")

        @pl.when(jnp.logical_and(cid == 0, sid == 0))
        def _():
            pltpu.sync_copy(idx_hbm, idx_v)
            for g in range(R // 16):
                ones_v[pl.ds(g * 16, 16)] = jnp.full((16,), 1.0, jnp.float32)
            pltpu.async_copy(ones_v, out_hbm.at[idx_v], sem).wait()

    return _sc_scatter_ones


def kernel(probs):
    zeros2d, idx2d = _argmax_zero(probs)
    buf = jax.new_ref(zeros2d.reshape((R * V,)))
    _make_sc_scatter_ones()(idx2d.reshape((R,)), buf)
    return buf[...].reshape((R, V))


# speedup vs baseline: 1.3382x; 1.3382x over previous
import functools

import jax
import jax.numpy as jnp
from jax.experimental import pallas as pl
from jax.experimental.pallas import tpu as pltpu
from jax.experimental.pallas import tpu_sc as plsc

R = 128
V = 100000
RB = 8
NSTEP = R // RB
ZB = RB * V


def _make_argmax_zero_body(buf):
    def _body(rb_ref, p_ref, idx_ref, zbuf, sem):
        r0 = rb_ref[0, 0, 0]

        @pl.when(r0 == 0)
        def _zinit():
            zbuf[...] = jnp.zeros((ZB,), jnp.float32)

        off = pl.multiple_of(r0 * V, 1024)
        pltpu.make_async_copy(zbuf, buf.at[pl.ds(off, ZB)], sem).start()

        x = p_ref[...]
        m = jnp.max(x, axis=1, keepdims=True)
        col = jax.lax.broadcasted_iota(jnp.int32, (RB, V), 1)
        cand = jnp.where(x == m, col, jnp.int32(2**31 - 1))
        first = jnp.min(cand, axis=1, keepdims=True)
        row = jax.lax.broadcasted_iota(jnp.int32, (RB, 1), 0) + r0
        idx_ref[...] = first + row * V

        @pl.when(r0 > 0)
        def _drain_prev():
            pltpu.make_async_copy(zbuf, buf.at[pl.ds(0, ZB)], sem).wait()

        @pl.when(r0 == R - RB)
        def _drain_last():
            pltpu.make_async_copy(zbuf, buf.at[pl.ds(0, ZB)], sem).wait()

    return _body


def _argmax_zero(probs, buf):
    row_base = (jnp.arange(NSTEP, dtype=jnp.int32) * RB).reshape(NSTEP, 1, 1)
    return pl.pallas_call(
        _make_argmax_zero_body(buf),
        grid=(NSTEP,),
        in_specs=[
            pl.BlockSpec((1, 1, 1), lambda j: (j, 0, 0), memory_space=pltpu.SMEM),
            pl.BlockSpec((RB, V), lambda j: (j, 0)),
        ],
        out_specs=pl.BlockSpec((RB, 1), lambda j: (j, 0)),
        out_shape=jax.ShapeDtypeStruct((R, 1), jnp.int32),
        scratch_shapes=[
            pltpu.VMEM((ZB,), jnp.float32),
            pltpu.SemaphoreType.DMA,
        ],
    )(row_base, probs)


@functools.cache
def _make_sc_scatter_ones():
    @functools.partial(
        pl.kernel,
        out_type=(),
        mesh=plsc.VectorSubcoreMesh(core_axis_name="c", subcore_axis_name="s"),
        scratch_types=[
            pltpu.VMEM((R,), jnp.int32),
            pltpu.VMEM((R,), jnp.float32),
            pltpu.SemaphoreType.DMA,
        ],
    )
    def _sc_scatter_ones(idx_hbm, out_hbm, idx_v, ones_v, sem):
        cid = jax.lax.axis_index("c")
        sid = jax.lax.axis_index("s")

        @pl.when(jnp.logical_and(cid == 0, sid == 0))
        def _():
            pltpu.sync_copy(idx_hbm, idx_v)
            for g in range(R // 16):
                ones_v[pl.ds(g * 16, 16)] = jnp.full((16,), 1.0, jnp.float32)
            pltpu.async_copy(ones_v, out_hbm.at[idx_v], sem).wait()

    return _sc_scatter_ones


def kernel(probs):
    buf = jax.empty_ref(jax.ShapeDtypeStruct((R * V,), jnp.float32))
    idx2d = _argmax_zero(probs, buf)
    _make_sc_scatter_ones()(idx2d.reshape((R,)), buf)
    return jax.freeze(buf).reshape((R, V))


# speedup vs baseline: 1.8845x; 1.4083x over previous
import functools

import jax
import jax.numpy as jnp
from jax.experimental import pallas as pl
from jax.experimental.pallas import tpu as pltpu
from jax.experimental.pallas import tpu_sc as plsc

R = 128
V = 100000
RB = 8
NSTEP = R // RB
ZC = 20000
NZC = V // ZC


def _argmax_body(p_ref, idx_ref):
    x = p_ref[...]
    m = jnp.max(x, axis=1, keepdims=True)
    col = jax.lax.broadcasted_iota(jnp.int32, (RB, V), 1)
    cand = jnp.where(x == m, col, jnp.int32(2**31 - 1))
    idx_ref[...] = jnp.min(cand, axis=1, keepdims=True)


_argmax = pl.pallas_call(
    _argmax_body,
    grid=(NSTEP,),
    in_specs=[pl.BlockSpec((RB, V), lambda j: (j, 0))],
    out_specs=pl.BlockSpec((RB, 1), lambda j: (j, 0)),
    out_shape=jax.ShapeDtypeStruct((R, 1), jnp.int32),
)


ZCA = 6400
NZA = 15
ZCB = V - NZA * ZCA


@functools.cache
def _make_sc_zero():
    info = plsc.get_sparse_core_info()

    @functools.partial(
        pl.kernel,
        out_type=(),
        mesh=plsc.VectorSubcoreMesh(core_axis_name="c", subcore_axis_name="s"),
        scratch_types=[
            pltpu.VMEM((RB, ZCA), jnp.float32),
            pltpu.VMEM((RB, ZCB), jnp.float32),
            pltpu.SemaphoreType.DMA,
        ],
    )
    def _sc_zero(out_hbm, zbufa, zbufb, sem):
        w = jax.lax.axis_index("s") * info.num_cores + jax.lax.axis_index("c")

        @pl.when(w < NSTEP)
        def _():
            for i in range(RB):
                def _zero_a(j, carry):
                    zbufa[i, pl.ds(j * 16, 16)] = jnp.zeros((16,), jnp.float32)
                    return carry
                jax.lax.fori_loop(0, ZCA // 16, _zero_a, 0)

                def _zero_b(j, carry):
                    zbufb[i, pl.ds(j * 16, 16)] = jnp.zeros((16,), jnp.float32)
                    return carry
                jax.lax.fori_loop(0, ZCB // 16, _zero_b, 0)

            r0 = w * RB
            copies = []
            for ch in range(NZA):
                cp = pltpu.make_async_copy(
                    zbufa,
                    out_hbm.at[pl.ds(r0, RB), pl.ds(ch * ZCA, ZCA)],
                    sem)
                cp.start()
                copies.append(cp)
            cpb = pltpu.make_async_copy(
                zbufb, out_hbm.at[pl.ds(r0, RB), pl.ds(NZA * ZCA, ZCB)], sem)
            cpb.start()
            for cp in copies:
                cp.wait()
            cpb.wait()

    return _sc_zero


def _make_patch_body(buf):
    def _body(idx_smem, idx_vmem, win, sem):
        c_vec = idx_vmem[...]
        col = jax.lax.broadcasted_iota(jnp.int32, (RB, 128), 1)
        for r in range(R):
            g = r // RB
            cs = pl.multiple_of((idx_smem[r, 0] // 128) * 128, 128)
            win[r] = jnp.where(
                col + cs == c_vec[g * RB:(g + 1) * RB], 1.0, 0.0)
        for r in range(R):
            g = r // RB
            cs = pl.multiple_of((idx_smem[r, 0] // 128) * 128, 128)
            pltpu.make_async_copy(
                win.at[r],
                buf.at[pl.ds(g * RB, RB), pl.ds(cs, 128)],
                sem,
            ).start()
        for r in range(R):
            pltpu.make_async_copy(
                win.at[r], buf.at[pl.ds(0, RB), pl.ds(0, 128)], sem).wait()

    return _body


def _patch(idx2d, buf):
    return pl.pallas_call(
        _make_patch_body(buf),
        in_specs=[
            pl.BlockSpec(memory_space=pltpu.SMEM),
            pl.BlockSpec(memory_space=pltpu.VMEM),
        ],
        out_specs=(),
        out_shape=(),
        scratch_shapes=[
            pltpu.VMEM((R, RB, 128), jnp.float32),
            pltpu.SemaphoreType.DMA,
        ],
    )(idx2d, idx2d)


def kernel(probs):
    buf = jax.empty_ref(jax.ShapeDtypeStruct((R, V), jnp.float32))
    _make_sc_zero()(buf)
    idx2d = _argmax(probs)
    _patch(idx2d, buf)
    return jax.freeze(buf)


# speedup vs baseline: 2.0448x; 1.0850x over previous
import functools

import jax
import jax.numpy as jnp
from jax.experimental import pallas as pl
from jax.experimental.pallas import tpu as pltpu
from jax.experimental.pallas import tpu_sc as plsc

R = 128
V = 100000
RB = 8
AB = 32
NSTEP = R // RB


def _argmax_body(p_ref, idx_ref):
    x = p_ref[...]
    m = jnp.max(x, axis=1, keepdims=True)
    col = jax.lax.broadcasted_iota(jnp.int32, (AB, V), 1)
    cand = jnp.where(x == m, col, jnp.int32(2**31 - 1))
    idx_ref[...] = jnp.min(cand, axis=1, keepdims=True)


_argmax = pl.pallas_call(
    _argmax_body,
    grid=(R // AB,),
    in_specs=[pl.BlockSpec((AB, V), lambda j: (j, 0))],
    out_specs=pl.BlockSpec((AB, 1), lambda j: (j, 0)),
    out_shape=jax.ShapeDtypeStruct((R, 1), jnp.int32),
)


ZCA = 6400
NZA = 15
ZCB = V - NZA * ZCA


@functools.cache
def _make_sc_zero():
    info = plsc.get_sparse_core_info()

    @functools.partial(
        pl.kernel,
        out_type=jax.ShapeDtypeStruct((R, V), jnp.float32),
        mesh=plsc.VectorSubcoreMesh(core_axis_name="c", subcore_axis_name="s"),
        scratch_types=[
            pltpu.VMEM((RB, ZCA), jnp.float32),
            pltpu.VMEM((RB, ZCB), jnp.float32),
            pltpu.SemaphoreType.DMA,
        ],
    )
    def _sc_zero(out_hbm, zbufa, zbufb, sem):
        w = jax.lax.axis_index("s") * info.num_cores + jax.lax.axis_index("c")

        @pl.when(w < NSTEP)
        def _():
            for i in range(RB):
                def _zero_a(j, carry):
                    zbufa[i, pl.ds(j * 16, 16)] = jnp.zeros((16,), jnp.float32)
                    return carry
                jax.lax.fori_loop(0, ZCA // 16, _zero_a, 0)

                def _zero_b(j, carry):
                    zbufb[i, pl.ds(j * 16, 16)] = jnp.zeros((16,), jnp.float32)
                    return carry
                jax.lax.fori_loop(0, ZCB // 16, _zero_b, 0)

            r0 = w * RB
            copies = []
            for ch in range(NZA):
                cp = pltpu.make_async_copy(
                    zbufa,
                    out_hbm.at[pl.ds(r0, RB), pl.ds(ch * ZCA, ZCA)],
                    sem)
                cp.start()
                copies.append(cp)
            cpb = pltpu.make_async_copy(
                zbufb, out_hbm.at[pl.ds(r0, RB), pl.ds(NZA * ZCA, ZCB)], sem)
            cpb.start()
            for cp in copies:
                cp.wait()
            cpb.wait()

    return _sc_zero


def _patch_body(idx_smem, idx_vmem, z_any, out_any, win, *sems):
    c_vec = idx_vmem[...]
    col = jax.lax.broadcasted_iota(jnp.int32, (RB, 128), 1)
    for r in range(R):
        g = r // RB
        cs = pl.multiple_of((idx_smem[r, 0] // 128) * 128, 128)
        win[r] = jnp.where(
            col + cs == c_vec[g * RB:(g + 1) * RB], 1.0, 0.0)
    for r in range(R):
        g = r // RB
        cs = pl.multiple_of((idx_smem[r, 0] // 128) * 128, 128)
        pltpu.make_async_copy(
            win.at[r],
            out_any.at[pl.ds(g * RB, RB), pl.ds(cs, 128)],
            sems[r % len(sems)],
        ).start()
    for r in range(R):
        pltpu.make_async_copy(
            win.at[r], out_any.at[pl.ds(0, RB), pl.ds(0, 128)],
            sems[r % len(sems)]).wait()


_patch = pl.pallas_call(
    _patch_body,
    in_specs=[
        pl.BlockSpec(memory_space=pltpu.SMEM),
        pl.BlockSpec(memory_space=pltpu.VMEM),
        pl.BlockSpec(memory_space=pl.ANY),
    ],
    out_specs=pl.BlockSpec(memory_space=pl.ANY),
    out_shape=jax.ShapeDtypeStruct((R, V), jnp.float32),
    scratch_shapes=[
        pltpu.VMEM((R, RB, 128), jnp.float32),
    ] + [pltpu.SemaphoreType.DMA] * 8,
    input_output_aliases={2: 0},
)


def kernel(probs):
    zeros2d = _make_sc_zero()()
    idx2d = _argmax(probs)
    return _patch(idx2d, idx2d, zeros2d)


# speedup vs baseline: 2.1332x; 1.0432x over previous
import jax
import jax.numpy as jnp
from jax.experimental import pallas as pl
from jax.experimental.pallas import tpu as pltpu

R = 128
V = 100000
RB = 8
NSTEP = R // RB


def _onehot_argmax_body(p_ref, o_ref):
    x = p_ref[...]
    m = jnp.max(x, axis=1, keepdims=True)
    col = jax.lax.broadcasted_iota(jnp.int32, (RB, V), 1)
    cand = jnp.where(x == m, col, jnp.int32(2**31 - 1))
    first = jnp.min(cand, axis=1, keepdims=True)
    o_ref[...] = (col == first).astype(jnp.float32)


_onehot_argmax = pl.pallas_call(
    _onehot_argmax_body,
    grid=(NSTEP,),
    in_specs=[pl.BlockSpec((RB, V), lambda j: (j, 0))],
    out_specs=pl.BlockSpec((RB, V), lambda j: (j, 0)),
    out_shape=jax.ShapeDtypeStruct((R, V), jnp.float32),
)


def kernel(probs):
    return _onehot_argmax(probs)


# speedup vs baseline: 5.3510x; 2.5084x over previous
import jax
import jax.numpy as jnp
from jax.experimental import pallas as pl
from jax.experimental.pallas import tpu as pltpu

R = 128
V = 100000
C = 10000
NSTEP = V // C


def _body(p_ref, o_ref, best_val, best_idx):
    ph = pl.program_id(0)
    j = pl.program_id(1)

    @pl.when(ph == 0)
    def _accumulate():
        @pl.when(j == 0)
        def _init():
            best_val[...] = jnp.full((1, R), -jnp.inf, jnp.float32)
            best_idx[...] = jnp.zeros((1, R), jnp.int32)

        x = p_ref[...]
        m = jnp.max(x, axis=0, keepdims=True)
        row = jax.lax.broadcasted_iota(jnp.int32, (C, R), 0)
        cand = jnp.where(x == m, row, jnp.int32(2**31 - 1))
        first = jnp.min(cand, axis=0, keepdims=True) + j * C
        upd = m > best_val[...]
        best_idx[...] = jnp.where(upd, first, best_idx[...])
        best_val[...] = jnp.where(upd, m, best_val[...])

    @pl.when(ph == 1)
    def _write():
        row = jax.lax.broadcasted_iota(jnp.int32, (C, R), 0) + j * C
        o_ref[...] = (row == best_idx[...]).astype(jnp.float32)


_onehot_argmax_t = pl.pallas_call(
    _body,
    grid=(2, NSTEP),
    in_specs=[pl.BlockSpec((C, R), lambda p, j: (j * (1 - p), 0))],
    out_specs=pl.BlockSpec((C, R), lambda p, j: (j, 0)),
    out_shape=jax.ShapeDtypeStruct((V, R), jnp.float32),
    scratch_shapes=[
        pltpu.VMEM((1, R), jnp.float32),
        pltpu.VMEM((1, R), jnp.int32),
    ],
)


def kernel(probs):
    return _onehot_argmax_t(probs.T).T


# speedup vs baseline: 6.4891x; 1.2127x over previous
import jax
import jax.numpy as jnp
from jax.experimental import pallas as pl
from jax.experimental.pallas import tpu as pltpu

R = 128
V = 100000
C = 10000
NSTEP = V // C


def _body(p_ref, o_ref, best_val, best_idx):
    ph = pl.program_id(0)
    j = pl.program_id(1)

    @pl.when(ph == 0)
    def _accumulate():
        @pl.when(j == 0)
        def _init():
            best_val[...] = jnp.full((1, R), -jnp.inf, jnp.float32)
            best_idx[...] = jnp.zeros((1, R), jnp.int32)

        x = p_ref[...]
        m = jnp.max(x, axis=0, keepdims=True)
        row = jax.lax.broadcasted_iota(jnp.int32, (C, R), 0)
        cand = jnp.where(x == m, row, jnp.int32(2**31 - 1))
        first = jnp.min(cand, axis=0, keepdims=True) + j * C
        upd = m > best_val[...]
        best_idx[...] = jnp.where(upd, first, best_idx[...])
        best_val[...] = jnp.where(upd, m, best_val[...])

    @pl.when(ph == 1)
    def _write():
        row = jax.lax.broadcasted_iota(jnp.int32, (C, R), 0) + j * C
        o_ref[...] = (row == best_idx[...]).astype(jnp.float32)


_onehot_argmax_t = pl.pallas_call(
    _body,
    grid=(2, NSTEP),
    in_specs=[pl.BlockSpec(
        (C, R), lambda p, j: (j * (1 - p) + (NSTEP - 1) * p, 0))],
    out_specs=pl.BlockSpec((C, R), lambda p, j: (j * p, 0)),
    out_shape=jax.ShapeDtypeStruct((V, R), jnp.float32),
    scratch_shapes=[
        pltpu.VMEM((1, R), jnp.float32),
        pltpu.VMEM((1, R), jnp.int32),
    ],
)


def kernel(probs):
    return _onehot_argmax_t(probs.T).T
